# baseline (speedup 1.0000x reference)
import functools
import math

import jax
import jax.numpy as jnp
from jax import lax
from jax.experimental import pallas as pl
from jax.experimental.pallas import tpu as pltpu
from jax.experimental.pallas import tpu_sc as plsc

BS, N, D = 32, 65536, 3
KEEP = math.ceil(N * 0.07)
NW = KEEP * D
CHUNK = 128
NCHUNK = -(-NW // CHUNK)
WPAD = NCHUNK * CHUNK

_NC = 2


@functools.lru_cache(maxsize=1)
def _word_indices():
    perm_key = jax.random.key(42)
    keys = jax.random.split(perm_key, BS)
    point_idxs = jnp.stack(
        [jax.random.permutation(k, N)[:KEEP] for k in keys]
    )
    flat_rows = point_idxs.astype(jnp.int32) + (
        jnp.arange(BS, dtype=jnp.int32) * N
    )[:, None]
    words = flat_rows[:, :, None] * D + jnp.arange(D, dtype=jnp.int32)
    words = words.reshape(BS, NW)
    pad = words[:, : WPAD - NW]
    return jnp.concatenate([words, pad], axis=1).reshape(BS, NCHUNK, CHUNK)


@functools.partial(
    pl.kernel,
    mesh=plsc.VectorSubcoreMesh(core_axis_name="c", subcore_axis_name="s"),
    out_type=jax.ShapeDtypeStruct((BS, WPAD), jnp.float32),
    scratch_types=[
        pltpu.VMEM((NCHUNK, CHUNK), jnp.int32),
        pltpu.VMEM((WPAD,), jnp.float32),
        pltpu.SemaphoreType.DMA,
    ],
)
def _gather_words(flat_hbm, idx_hbm, out_hbm, idx_v, words_v, sem):
    w = lax.axis_index("s") * _NC + lax.axis_index("c")
    pltpu.sync_copy(idx_hbm.at[w], idx_v)

    def chunk(j, carry):
        pltpu.async_copy(
            flat_hbm.at[idx_v.at[j]], words_v.at[pl.ds(j * CHUNK, CHUNK)], sem
        ).wait()
        return carry

    lax.fori_loop(0, NCHUNK, chunk, 0)
    pltpu.sync_copy(words_v, out_hbm.at[w])


def kernel(pc):
    flat = pc.reshape(BS * N * D)
    padded = _gather_words(flat, _word_indices())
    return padded[:, :NW].reshape(BS, KEEP, D)


# speedup vs baseline: 1.0081x; 1.0081x over previous
import functools
import math

import jax
import jax.numpy as jnp
from jax import lax
from jax.experimental import pallas as pl
from jax.experimental.pallas import tpu as pltpu
from jax.experimental.pallas import tpu_sc as plsc

BS, N, D = 32, 65536, 3
KEEP = math.ceil(N * 0.07)
NW = KEEP * D
CHUNK = 128
NCHUNK = -(-NW // CHUNK)
WPAD = NCHUNK * CHUNK

_NC = 2


@functools.lru_cache(maxsize=1)
def _word_indices():
    perm_key = jax.random.key(42)
    keys = jax.random.split(perm_key, BS)
    point_idxs = jnp.stack(
        [jax.random.permutation(k, N)[:KEEP] for k in keys]
    )
    flat_rows = point_idxs.astype(jnp.int32) + (
        jnp.arange(BS, dtype=jnp.int32) * N
    )[:, None]
    words = flat_rows[:, :, None] * D + jnp.arange(D, dtype=jnp.int32)
    words = words.reshape(BS, NW)
    pad = words[:, : WPAD - NW]
    return jnp.concatenate([words, pad], axis=1)


@functools.partial(
    pl.kernel,
    mesh=plsc.VectorSubcoreMesh(core_axis_name="c", subcore_axis_name="s"),
    out_type=jax.ShapeDtypeStruct((BS, WPAD), jnp.float32),
    scratch_types=[
        pltpu.VMEM((WPAD,), jnp.int32),
        pltpu.VMEM((WPAD,), jnp.float32),
        pltpu.SemaphoreType.DMA,
    ],
)
def _gather_words(flat_hbm, idx_hbm, out_hbm, idx_v, words_v, sem):
    w = lax.axis_index("s") * _NC + lax.axis_index("c")
    pltpu.sync_copy(idx_hbm.at[w], idx_v)
    pltpu.async_copy(flat_hbm.at[idx_v], words_v, sem).wait()
    pltpu.sync_copy(words_v, out_hbm.at[w])


def kernel(pc):
    flat = pc.reshape(BS * N * D)
    padded = _gather_words(flat, _word_indices())
    return padded[:, :NW].reshape(BS, KEEP, D)
